# baseline (speedup 1.0000x reference)
import functools

import jax
import jax.numpy as jnp
from jax import lax
from jax.experimental import pallas as pl
from jax.experimental.pallas import tpu as pltpu
from jax.experimental.pallas import tpu_sc as plsc

_ORG_VOCAB = 100000
_EXTRA_VOCAB = 256
_EMBED_DIM = 2048
_MAX_LORAS = 8
_RANK = 16
_PACK = 128 // _RANK

_NC, _NS = 2, 16
_NW = _NC * _NS
_CHUNK = 16
_ACHUNK = 128


def _sc_gather_build(n_tok: int):
    tpw = n_tok // _NW
    nch = tpw // _CHUNK
    nach = tpw // _ACHUNK
    mesh = plsc.VectorSubcoreMesh(core_axis_name="c", subcore_axis_name="s")

    @functools.partial(
        pl.kernel,
        out_type=[
            jax.ShapeDtypeStruct((n_tok, _EMBED_DIM), jnp.float32),
            jax.ShapeDtypeStruct((n_tok, _PACK * _RANK), jnp.float32),
        ],
        mesh=mesh,
        compiler_params=pltpu.CompilerParams(use_tc_tiling_on_sc=True),
        scratch_types=[
            pltpu.VMEM((tpw,), jnp.int32),
            pltpu.VMEM((tpw,), jnp.int32),
            pltpu.VMEM((tpw, _PACK * _RANK), jnp.float32),
            pltpu.VMEM((_CHUNK, _EMBED_DIM), jnp.float32),
            pltpu.VMEM((_CHUNK, _EMBED_DIM), jnp.float32),
            pltpu.SemaphoreType.DMA,
            pltpu.SemaphoreType.DMA,
            pltpu.SemaphoreType.DMA,
        ],
    )
    def sc_gather(base_hbm, lora_a_hbm, idx_hbm, arow_hbm, rows_out, a_out,
                  idx_v, arow_v, a_v, buf0, buf1, sem0, sem1, sem_a):
        wid = lax.axis_index("s") * _NC + lax.axis_index("c")
        base = wid * tpw
        pltpu.sync_copy(idx_hbm.at[pl.ds(base, tpw)], idx_v)
        pltpu.sync_copy(arow_hbm.at[pl.ds(base, tpw)], arow_v)
        a_dmas = []
        for c in range(nach):
            a_dmas.append(pltpu.async_copy(
                lora_a_hbm.at[arow_v.at[pl.ds(c * _ACHUNK, _ACHUNK)]],
                a_v.at[pl.ds(c * _ACHUNK, _ACHUNK)], sem_a))

        bufs = (buf0, buf1)
        sems = (sem0, sem1)
        dmas = [None, None]
        dmas[0] = pltpu.async_copy(
            base_hbm.at[idx_v.at[pl.ds(0, _CHUNK)]], bufs[0], sems[0])
        for c in range(nch):
            nxt = c + 1
            if nxt < nch:
                dmas[nxt % 2] = pltpu.async_copy(
                    base_hbm.at[idx_v.at[pl.ds(nxt * _CHUNK, _CHUNK)]],
                    bufs[nxt % 2], sems[nxt % 2])
            dmas[c % 2].wait()
            pltpu.sync_copy(bufs[c % 2],
                            rows_out.at[pl.ds(base + c * _CHUNK, _CHUNK)])
        for d in a_dmas:
            d.wait()
        pltpu.sync_copy(a_v, a_out.at[pl.ds(base, tpw)])

    return sc_gather


def _tc_body(rows_ref, aw_ref, sub_ref, idx_ref, bt_ref, out_ref):
    aw = aw_ref[...]
    sub = sub_ref[...]
    idx = idx_ref[...]
    bt_blk = aw.shape[0]
    a_sel = jnp.zeros((bt_blk, _RANK), jnp.float32)
    for g in range(_PACK):
        a_sel = a_sel + jnp.where(
            sub == g, aw[:, g * _RANK:(g + 1) * _RANK], 0.0)
    cols = lax.broadcasted_iota(jnp.int32, (bt_blk, _MAX_LORAS * _RANK), 1)
    sel = (cols // _RANK) == idx
    a_exp = jnp.where(sel, jnp.concatenate([a_sel] * _MAX_LORAS, axis=1), 0.0)
    delta = jnp.dot(a_exp, bt_ref[...],
                    preferred_element_type=jnp.float32,
                    precision=lax.Precision.HIGHEST)
    out_ref[...] = rows_ref[...] + delta


def kernel(x, base_weight, lora_a_stacked, lora_b_stacked, base_indices,
           embeddings_indices):
    b, s = x.shape
    n_tok = b * s
    xf = x.reshape(n_tok).astype(jnp.int32)
    aidx = xf + embeddings_indices[1][:n_tok]
    arow = aidx >> 3
    sub2 = (aidx & (_PACK - 1)).reshape(n_tok, 1)

    lora_a_packed = lora_a_stacked.reshape(
        _MAX_LORAS * (_ORG_VOCAB + _EXTRA_VOCAB) // _PACK, _PACK * _RANK)
    bt2 = lora_b_stacked[:, 0].transpose(0, 2, 1).reshape(
        _MAX_LORAS * _RANK, _EMBED_DIM)

    rows, a_wide = _sc_gather_build(n_tok)(base_weight, lora_a_packed, xf, arow)

    bt_tok = 512
    grid = (n_tok // bt_tok,)
    idx2 = base_indices[:n_tok].reshape(n_tok, 1).astype(jnp.int32)
    out = pl.pallas_call(
        _tc_body,
        grid=grid,
        in_specs=[
            pl.BlockSpec((bt_tok, _EMBED_DIM), lambda i: (i, 0)),
            pl.BlockSpec((bt_tok, _PACK * _RANK), lambda i: (i, 0)),
            pl.BlockSpec((bt_tok, 1), lambda i: (i, 0)),
            pl.BlockSpec((bt_tok, 1), lambda i: (i, 0)),
            pl.BlockSpec((_MAX_LORAS * _RANK, _EMBED_DIM), lambda i: (0, 0)),
        ],
        out_specs=pl.BlockSpec((bt_tok, _EMBED_DIM), lambda i: (i, 0)),
        out_shape=jax.ShapeDtypeStruct((n_tok, _EMBED_DIM), jnp.float32),
    )(rows, a_wide, sub2, idx2, bt2)

    return out.reshape(b, s, _EMBED_DIM)


# speedup vs baseline: 2.3042x; 2.3042x over previous
import functools

import jax
import jax.numpy as jnp
from jax import lax
from jax.experimental import pallas as pl
from jax.experimental.pallas import tpu as pltpu
from jax.experimental.pallas import tpu_sc as plsc

_ORG_VOCAB = 100000
_EXTRA_VOCAB = 256
_EMBED_DIM = 2048
_MAX_LORAS = 8
_RANK = 16

_NC, _NS = 2, 16
_NW = _NC * _NS
_CHUNK = 16


def _sc_gather_build(n_tok: int):
    tpw = n_tok // _NW
    nch = tpw // _CHUNK
    mesh = plsc.VectorSubcoreMesh(core_axis_name="c", subcore_axis_name="s")

    @functools.partial(
        pl.kernel,
        out_type=[
            jax.ShapeDtypeStruct((n_tok, _EMBED_DIM), jnp.float32),
            jax.ShapeDtypeStruct((n_tok, _RANK), jnp.float32),
        ],
        mesh=mesh,
        scratch_types=[
            pltpu.VMEM((tpw,), jnp.int32),
            pltpu.VMEM((tpw,), jnp.int32),
            pltpu.VMEM((tpw, _RANK), jnp.float32),
            pltpu.VMEM((_CHUNK, _EMBED_DIM), jnp.float32),
            pltpu.VMEM((_CHUNK, _EMBED_DIM), jnp.float32),
            pltpu.SemaphoreType.DMA,
            pltpu.SemaphoreType.DMA,
            pltpu.SemaphoreType.DMA,
        ],
    )
    def sc_gather(base_hbm, lora_a_hbm, idx_hbm, aidx_hbm, rows_out, a_out,
                  idx_v, aidx_v, a_v, buf0, buf1, sem0, sem1, sem_a):
        wid = lax.axis_index("s") * _NC + lax.axis_index("c")
        base = wid * tpw
        pltpu.sync_copy(idx_hbm.at[pl.ds(base, tpw)], idx_v)
        pltpu.sync_copy(aidx_hbm.at[pl.ds(base, tpw)], aidx_v)

        bufs = (buf0, buf1)
        sems = (sem0, sem1)
        dmas = [None, None]
        dmas[0] = pltpu.async_copy(
            base_hbm.at[idx_v.at[pl.ds(0, _CHUNK)]], bufs[0], sems[0])
        a_pend, a_prev = [], []
        for c in range(nch):
            a_pend = []
            aidx_vec = aidx_v[pl.ds(c * _CHUNK, _CHUNK)]
            for j in range(_CHUNK):
                t = c * _CHUNK + j
                r = aidx_vec[j]
                a_pend.append(pltpu.async_copy(
                    lora_a_hbm.at[pl.ds(r, 1)], a_v.at[pl.ds(t, 1)], sem_a))
            nxt = c + 1
            if nxt < nch:
                dmas[nxt % 2] = pltpu.async_copy(
                    base_hbm.at[idx_v.at[pl.ds(nxt * _CHUNK, _CHUNK)]],
                    bufs[nxt % 2], sems[nxt % 2])
            dmas[c % 2].wait()
            pltpu.sync_copy(bufs[c % 2],
                            rows_out.at[pl.ds(base + c * _CHUNK, _CHUNK)])
            for h in a_prev:
                h.wait()
            a_prev = a_pend
        for h in a_prev:
            h.wait()
        pltpu.sync_copy(a_v, a_out.at[pl.ds(base, tpw)])

    return sc_gather


def _tc_body(rows_ref, a_ref, idx_ref, bt_ref, out_ref):
    a = a_ref[...]
    idx = idx_ref[...]
    bt_blk = a.shape[0]
    cols = lax.broadcasted_iota(jnp.int32, (bt_blk, _MAX_LORAS * _RANK), 1)
    sel = (cols // _RANK) == idx
    a_exp = jnp.where(sel, jnp.concatenate([a] * _MAX_LORAS, axis=1), 0.0)
    delta = jnp.dot(a_exp, bt_ref[...],
                    preferred_element_type=jnp.float32,
                    precision=lax.Precision.HIGHEST)
    out_ref[...] = rows_ref[...] + delta


def kernel(x, base_weight, lora_a_stacked, lora_b_stacked, base_indices,
           embeddings_indices):
    b, s = x.shape
    n_tok = b * s
    xf = x.reshape(n_tok).astype(jnp.int32)
    aidx = xf + embeddings_indices[1][:n_tok]

    lora_a_2d = lora_a_stacked.reshape(
        _MAX_LORAS * (_ORG_VOCAB + _EXTRA_VOCAB), _RANK)
    bt2 = lora_b_stacked[:, 0].transpose(0, 2, 1).reshape(
        _MAX_LORAS * _RANK, _EMBED_DIM)

    rows, a_rows = _sc_gather_build(n_tok)(base_weight, lora_a_2d, xf, aidx)

    bt_tok = 512
    grid = (n_tok // bt_tok,)
    idx2 = base_indices[:n_tok].reshape(n_tok, 1).astype(jnp.int32)
    out = pl.pallas_call(
        _tc_body,
        grid=grid,
        in_specs=[
            pl.BlockSpec((bt_tok, _EMBED_DIM), lambda i: (i, 0)),
            pl.BlockSpec((bt_tok, _RANK), lambda i: (i, 0)),
            pl.BlockSpec((bt_tok, 1), lambda i: (i, 0)),
            pl.BlockSpec((_MAX_LORAS * _RANK, _EMBED_DIM), lambda i: (0, 0)),
        ],
        out_specs=pl.BlockSpec((bt_tok, _EMBED_DIM), lambda i: (i, 0)),
        out_shape=jax.ShapeDtypeStruct((n_tok, _EMBED_DIM), jnp.float32),
    )(rows, a_rows, idx2, bt2)

    return out.reshape(b, s, _EMBED_DIM)


# speedup vs baseline: 2.4753x; 1.0742x over previous
import functools

import jax
import jax.numpy as jnp
from jax import lax
from jax.experimental import pallas as pl
from jax.experimental.pallas import tpu as pltpu
from jax.experimental.pallas import tpu_sc as plsc

_ORG_VOCAB = 100000
_EXTRA_VOCAB = 256
_EMBED_DIM = 2048
_MAX_LORAS = 8
_RANK = 16

_NC, _NS = 2, 16
_NW = _NC * _NS
_CHUNK = 16


def _sc_base_build(n_tok: int):
    tpw = n_tok // _NW
    nch = tpw // _CHUNK
    mesh = plsc.VectorSubcoreMesh(core_axis_name="c", subcore_axis_name="s")

    @functools.partial(
        pl.kernel,
        out_type=jax.ShapeDtypeStruct((n_tok, _EMBED_DIM), jnp.float32),
        mesh=mesh,
        scratch_types=[
            pltpu.VMEM((tpw,), jnp.int32),
            pltpu.VMEM((_CHUNK, _EMBED_DIM), jnp.float32),
            pltpu.VMEM((_CHUNK, _EMBED_DIM), jnp.float32),
            pltpu.SemaphoreType.DMA,
            pltpu.SemaphoreType.DMA,
        ],
    )
    def sc_base(base_hbm, idx_hbm, rows_out, idx_v, buf0, buf1, sem0, sem1):
        wid = lax.axis_index("s") * _NC + lax.axis_index("c")
        base = wid * tpw
        pltpu.sync_copy(idx_hbm.at[pl.ds(base, tpw)], idx_v)
        bufs = (buf0, buf1)
        sems = (sem0, sem1)
        dmas = [None, None]
        dmas[0] = pltpu.async_copy(
            base_hbm.at[idx_v.at[pl.ds(0, _CHUNK)]], bufs[0], sems[0])
        for c in range(nch):
            nxt = c + 1
            if nxt < nch:
                dmas[nxt % 2] = pltpu.async_copy(
                    base_hbm.at[idx_v.at[pl.ds(nxt * _CHUNK, _CHUNK)]],
                    bufs[nxt % 2], sems[nxt % 2])
            dmas[c % 2].wait()
            pltpu.sync_copy(bufs[c % 2],
                            rows_out.at[pl.ds(base + c * _CHUNK, _CHUNK)])

    return sc_base


def _sc_lora_a_build(n_tok: int):
    tpw = n_tok // _NW
    mesh = plsc.VectorSubcoreMesh(core_axis_name="c", subcore_axis_name="s")

    @functools.partial(
        pl.kernel,
        out_type=jax.ShapeDtypeStruct((n_tok, _RANK), jnp.float32),
        mesh=mesh,
        scratch_types=[
            pltpu.VMEM((tpw,), jnp.int32),
            pltpu.VMEM((tpw, _RANK), jnp.float32),
            pltpu.SemaphoreType.DMA,
        ],
    )
    def sc_lora_a(lora_a_hbm, aidx_hbm, a_out, aidx_v, a_v, sem_a):
        wid = lax.axis_index("s") * _NC + lax.axis_index("c")
        base = wid * tpw
        pltpu.sync_copy(aidx_hbm.at[pl.ds(base, tpw)], aidx_v)
        pend, prev = [], []
        for g in range(tpw // 16):
            aidx_vec = aidx_v[pl.ds(g * 16, 16)]
            pend = []
            for j in range(16):
                t = g * 16 + j
                pend.append(pltpu.async_copy(
                    lora_a_hbm.at[pl.ds(aidx_vec[j], 1)],
                    a_v.at[pl.ds(t, 1)], sem_a))
            for h in prev:
                h.wait()
            prev = pend
        for h in prev:
            h.wait()
        pltpu.sync_copy(a_v, a_out.at[pl.ds(base, tpw)])

    return sc_lora_a


def _tc_body(rows_ref, a_ref, idx_ref, bt_ref, out_ref):
    a = a_ref[...]
    idx = idx_ref[...]
    bt_blk = a.shape[0]
    cols = lax.broadcasted_iota(jnp.int32, (bt_blk, _MAX_LORAS * _RANK), 1)
    sel = (cols // _RANK) == idx
    a_exp = jnp.where(sel, jnp.concatenate([a] * _MAX_LORAS, axis=1), 0.0)
    delta = jnp.dot(a_exp.astype(jnp.bfloat16), bt_ref[...],
                    preferred_element_type=jnp.float32)
    out_ref[...] = rows_ref[...] + delta


def kernel(x, base_weight, lora_a_stacked, lora_b_stacked, base_indices,
           embeddings_indices):
    b, s = x.shape
    n_tok = b * s
    xf = x.reshape(n_tok).astype(jnp.int32)
    aidx = xf + embeddings_indices[1][:n_tok]

    lora_a_2d = lora_a_stacked.reshape(
        _MAX_LORAS * (_ORG_VOCAB + _EXTRA_VOCAB), _RANK)
    bt2 = lora_b_stacked[:, 0].transpose(0, 2, 1).reshape(
        _MAX_LORAS * _RANK, _EMBED_DIM).astype(jnp.bfloat16)

    rows = _sc_base_build(n_tok)(base_weight, xf)
    a_rows = _sc_lora_a_build(n_tok)(lora_a_2d, aidx)

    bt_tok = 512
    grid = (n_tok // bt_tok,)
    idx2 = base_indices[:n_tok].reshape(n_tok, 1).astype(jnp.int32)
    out = pl.pallas_call(
        _tc_body,
        grid=grid,
        in_specs=[
            pl.BlockSpec((bt_tok, _EMBED_DIM), lambda i: (i, 0)),
            pl.BlockSpec((bt_tok, _RANK), lambda i: (i, 0)),
            pl.BlockSpec((bt_tok, 1), lambda i: (i, 0)),
            pl.BlockSpec((_MAX_LORAS * _RANK, _EMBED_DIM), lambda i: (0, 0)),
        ],
        out_specs=pl.BlockSpec((bt_tok, _EMBED_DIM), lambda i: (i, 0)),
        out_shape=jax.ShapeDtypeStruct((n_tok, _EMBED_DIM), jnp.float32),
    )(rows, a_rows, idx2, bt2)

    return out.reshape(b, s, _EMBED_DIM)


# speedup vs baseline: 2.9782x; 1.2032x over previous
import functools

import jax
import jax.numpy as jnp
from jax import lax
from jax.experimental import pallas as pl
from jax.experimental.pallas import tpu as pltpu
from jax.experimental.pallas import tpu_sc as plsc

_ORG_VOCAB = 100000
_EXTRA_VOCAB = 256
_EMBED_DIM = 2048
_MAX_LORAS = 8
_RANK = 16

_NC, _NS = 2, 16
_NW = _NC * _NS
_CHUNK = 16


def _sc_base_build(n_tok: int):
    tpw = n_tok // _NW
    nch = tpw // _CHUNK
    mesh = plsc.VectorSubcoreMesh(core_axis_name="c", subcore_axis_name="s")

    @functools.partial(
        pl.kernel,
        out_type=jax.ShapeDtypeStruct((n_tok, _EMBED_DIM), jnp.float32),
        mesh=mesh,
        scratch_types=[
            pltpu.VMEM((tpw,), jnp.int32),
            pltpu.VMEM((_CHUNK, _EMBED_DIM), jnp.float32),
            pltpu.VMEM((_CHUNK, _EMBED_DIM), jnp.float32),
            pltpu.SemaphoreType.DMA,
            pltpu.SemaphoreType.DMA,
        ],
    )
    def sc_base(base_hbm, idx_hbm, rows_out, idx_v, buf0, buf1, sem0, sem1):
        wid = lax.axis_index("s") * _NC + lax.axis_index("c")
        base = wid * tpw
        pltpu.sync_copy(idx_hbm.at[pl.ds(base, tpw)], idx_v)
        bufs = (buf0, buf1)
        sems = (sem0, sem1)
        dmas = [None, None]
        dmas[0] = pltpu.async_copy(
            base_hbm.at[idx_v.at[pl.ds(0, _CHUNK)]], bufs[0], sems[0])
        for c in range(nch):
            nxt = c + 1
            if nxt < nch:
                dmas[nxt % 2] = pltpu.async_copy(
                    base_hbm.at[idx_v.at[pl.ds(nxt * _CHUNK, _CHUNK)]],
                    bufs[nxt % 2], sems[nxt % 2])
            dmas[c % 2].wait()
            pltpu.sync_copy(bufs[c % 2],
                            rows_out.at[pl.ds(base + c * _CHUNK, _CHUNK)])

    return sc_base


def _sc_lora_a_build(n_tok: int):
    tpw = n_tok // _NW
    grp = 16
    ngr = tpw // grp
    mesh = plsc.VectorSubcoreMesh(core_axis_name="c", subcore_axis_name="s")

    @functools.partial(
        pl.kernel,
        out_type=jax.ShapeDtypeStruct((_RANK, n_tok), jnp.float32),
        mesh=mesh,
        compiler_params=pltpu.CompilerParams(needs_layout_passes=False),
        scratch_types=[
            pltpu.VMEM((tpw,), jnp.int32),
            pltpu.VMEM((tpw,), jnp.int32),
            pltpu.VMEM((_RANK, tpw), jnp.float32),
            pltpu.VMEM((grp, _RANK, 128), jnp.float32),
            pltpu.VMEM((grp, _RANK, 128), jnp.float32),
            pltpu.SemaphoreType.DMA,
            pltpu.SemaphoreType.DMA,
        ],
    )
    def sc_lora_a(at_hbm, lidx_hbm, v_hbm, at_out, lidx_v, v_v, a_vt,
                  slab0, slab1, sem0, sem1):
        wid = lax.axis_index("s") * _NC + lax.axis_index("c")
        base = wid * tpw
        pltpu.sync_copy(lidx_hbm.at[pl.ds(base, tpw)], lidx_v)
        pltpu.sync_copy(v_hbm.at[pl.ds(base, tpw)], v_v)
        row = lax.iota(jnp.int32, 16)

        def fire(g, buf, sem):
            l_vec = lidx_v[pl.ds(g * grp, grp)]
            v_vec = v_v[pl.ds(g * grp, grp)]
            for j in range(grp):
                v0 = pl.multiple_of((v_vec[j] >> 7) << 7, 128)
                pltpu.async_copy(
                    at_hbm.at[l_vec[j], :, pl.ds(v0, 128)], buf.at[j], sem)

        def wait_all(buf, sem):
            for j in range(grp):
                pltpu.make_async_copy(
                    at_hbm.at[0, :, pl.ds(0, 128)], buf.at[j], sem).wait()

        def extract(g, buf):
            v_vec = v_v[pl.ds(g * grp, grp)]
            for j in range(grp):
                t = g * grp + j
                col = jnp.full((16,), v_vec[j] & 127, jnp.int32)
                val = plsc.load_gather(
                    buf, [jnp.full((16,), j, jnp.int32), row, col])
                plsc.store_scatter(
                    a_vt, [row, jnp.full((16,), t, jnp.int32)], val)

        half = ngr // 2
        fire(0, slab0, sem0)

        def body(i, carry):
            g0 = 2 * i
            fire(g0 + 1, slab1, sem1)
            wait_all(slab0, sem0)
            extract(g0, slab0)

            @pl.when(i + 1 < half)
            def _():
                fire(g0 + 2, slab0, sem0)

            wait_all(slab1, sem1)
            extract(g0 + 1, slab1)
            return carry

        lax.fori_loop(0, half, body, 0)
        pltpu.sync_copy(a_vt, at_out.at[:, pl.ds(base, tpw)])

    return sc_lora_a


def _tc_body(rows_ref, at_ref, idx_ref, bt_ref, out_ref):
    a = jnp.transpose(at_ref[...], (1, 0))
    idx = idx_ref[...]
    bt_blk = a.shape[0]
    cols = lax.broadcasted_iota(jnp.int32, (bt_blk, _MAX_LORAS * _RANK), 1)
    sel = (cols // _RANK) == idx
    a_exp = jnp.where(sel, jnp.concatenate([a] * _MAX_LORAS, axis=1), 0.0)
    delta = jnp.dot(a_exp.astype(jnp.bfloat16), bt_ref[...],
                    preferred_element_type=jnp.float32)
    out_ref[...] = rows_ref[...] + delta


def kernel(x, base_weight, lora_a_stacked, lora_b_stacked, base_indices,
           embeddings_indices):
    b, s = x.shape
    n_tok = b * s
    xf = x.reshape(n_tok).astype(jnp.int32)
    lidx = (embeddings_indices[1][:n_tok]
            // (_ORG_VOCAB + _EXTRA_VOCAB)).astype(jnp.int32)

    at3 = jnp.transpose(lora_a_stacked, (0, 2, 1))
    bt2 = lora_b_stacked[:, 0].transpose(0, 2, 1).reshape(
        _MAX_LORAS * _RANK, _EMBED_DIM).astype(jnp.bfloat16)

    rows = _sc_base_build(n_tok)(base_weight, xf)
    a_rows = _sc_lora_a_build(n_tok)(at3, lidx, xf)

    bt_tok = 512
    grid = (n_tok // bt_tok,)
    idx2 = base_indices[:n_tok].reshape(n_tok, 1).astype(jnp.int32)
    out = pl.pallas_call(
        _tc_body,
        grid=grid,
        in_specs=[
            pl.BlockSpec((bt_tok, _EMBED_DIM), lambda i: (i, 0)),
            pl.BlockSpec((_RANK, bt_tok), lambda i: (0, i)),
            pl.BlockSpec((bt_tok, 1), lambda i: (i, 0)),
            pl.BlockSpec((_MAX_LORAS * _RANK, _EMBED_DIM), lambda i: (0, 0)),
        ],
        out_specs=pl.BlockSpec((bt_tok, _EMBED_DIM), lambda i: (i, 0)),
        out_shape=jax.ShapeDtypeStruct((n_tok, _EMBED_DIM), jnp.float32),
    )(rows, a_rows, idx2, bt2)

    return out.reshape(b, s, _EMBED_DIM)


# speedup vs baseline: 3.2358x; 1.0865x over previous
import functools

import jax
import jax.numpy as jnp
from jax import lax
from jax.experimental import pallas as pl
from jax.experimental.pallas import tpu as pltpu
from jax.experimental.pallas import tpu_sc as plsc

_ORG_VOCAB = 100000
_EXTRA_VOCAB = 256
_EMBED_DIM = 2048
_MAX_LORAS = 8
_RANK = 16

_NC, _NS = 2, 16
_NW = _NC * _NS
_CHUNK = 8
_AGRP = 8


def _sc_gather_build(n_tok: int):
    tpw = n_tok // _NW
    nch = tpw // _CHUNK
    ngr = tpw // _AGRP
    assert ngr == nch
    mesh = plsc.VectorSubcoreMesh(core_axis_name="c", subcore_axis_name="s")

    @functools.partial(
        pl.kernel,
        out_type=[
            jax.ShapeDtypeStruct((n_tok, _EMBED_DIM), jnp.float32),
            jax.ShapeDtypeStruct((_RANK, n_tok), jnp.float32),
        ],
        mesh=mesh,
        compiler_params=pltpu.CompilerParams(needs_layout_passes=False),
        scratch_types=[
            pltpu.VMEM((tpw,), jnp.int32),
            pltpu.VMEM((tpw,), jnp.int32),
            pltpu.VMEM((_RANK, tpw), jnp.float32),
            pltpu.VMEM((_CHUNK, _EMBED_DIM), jnp.float32),
            pltpu.VMEM((_CHUNK, _EMBED_DIM), jnp.float32),
            pltpu.VMEM((_AGRP, _RANK, 128), jnp.float32),
            pltpu.VMEM((_AGRP, _RANK, 128), jnp.float32),
            pltpu.SemaphoreType.DMA,
            pltpu.SemaphoreType.DMA,
            pltpu.SemaphoreType.DMA,
            pltpu.SemaphoreType.DMA,
        ],
    )
    def sc_gather(base_hbm, at_hbm, idx_hbm, lidx_hbm, rows_out, at_out,
                  idx_v, lidx_v, a_vt, buf0, buf1, slab0, slab1,
                  sem0, sem1, sema0, sema1):
        wid = lax.axis_index("s") * _NC + lax.axis_index("c")
        base = wid * tpw
        pltpu.sync_copy(idx_hbm.at[pl.ds(base, tpw)], idx_v)
        pltpu.sync_copy(lidx_hbm.at[pl.ds(base, tpw)], lidx_v)
        row = lax.iota(jnp.int32, 16)
        slabs = (slab0, slab1)
        semas = (sema0, sema1)

        bufs = (buf0, buf1)
        sems = (sem0, sem1)

        def fire_base(c, k):
            pltpu.async_copy(
                base_hbm.at[idx_v.at[pl.ds(c * _CHUNK, _CHUNK)]],
                bufs[k], sems[k])

        def wait_base(k):
            pltpu.make_async_copy(
                base_hbm.at[idx_v.at[pl.ds(0, _CHUNK)]],
                bufs[k], sems[k]).wait()

        def fire_a(g, k):
            off = (g // 2) * 16
            l_vec = lidx_v[pl.ds(off, 16)]
            v_vec = idx_v[pl.ds(off, 16)]
            for j in range(_AGRP):
                jj = j + _AGRP * k
                v0 = pl.multiple_of((v_vec[jj] >> 7) << 7, 128)
                pltpu.async_copy(
                    at_hbm.at[l_vec[jj], :, pl.ds(v0, 128)],
                    slabs[k].at[j], semas[k])

        def wait_a(k):
            for j in range(_AGRP):
                pltpu.make_async_copy(
                    at_hbm.at[0, :, pl.ds(0, 128)],
                    slabs[k].at[j], semas[k]).wait()

        def extract_a(g, k):
            v_vec = idx_v[pl.ds((g // 2) * 16, 16)]
            for j in range(_AGRP):
                jj = j + _AGRP * k
                t = g * _AGRP + j
                col = jnp.full((16,), v_vec[jj] & 127, jnp.int32)
                val = plsc.load_gather(
                    slabs[k], [jnp.full((16,), j, jnp.int32), row, col])
                plsc.store_scatter(
                    a_vt, [row, jnp.full((16,), t, jnp.int32)], val)

        half = nch // 2
        fire_base(0, 0)
        fire_a(0, 0)

        def body(i, carry):
            c0 = 2 * i
            fire_base(c0 + 1, 1)
            fire_a(c0 + 1, 1)
            wait_base(0)
            pltpu.sync_copy(bufs[0],
                            rows_out.at[pl.ds(base + c0 * _CHUNK, _CHUNK)])
            wait_a(0)
            extract_a(c0, 0)

            @pl.when(i + 1 < half)
            def _():
                fire_base(c0 + 2, 0)
                fire_a(c0 + 2, 0)

            wait_base(1)
            pltpu.sync_copy(
                bufs[1],
                rows_out.at[pl.ds(base + (c0 + 1) * _CHUNK, _CHUNK)])
            wait_a(1)
            extract_a(c0 + 1, 1)
            return carry

        lax.fori_loop(0, half, body, 0)
        pltpu.sync_copy(a_vt, at_out.at[:, pl.ds(base, tpw)])

    return sc_gather


def _tc_body(rows_ref, at_ref, idx_ref, bt_ref, out_ref):
    a = jnp.transpose(at_ref[...], (1, 0))
    idx = idx_ref[...]
    bt_blk = a.shape[0]
    cols = lax.broadcasted_iota(jnp.int32, (bt_blk, _MAX_LORAS * _RANK), 1)
    sel = (cols // _RANK) == idx
    a_exp = jnp.where(sel, jnp.concatenate([a] * _MAX_LORAS, axis=1), 0.0)
    delta = jnp.dot(a_exp.astype(jnp.bfloat16), bt_ref[...],
                    preferred_element_type=jnp.float32)
    out_ref[...] = rows_ref[...] + delta


def kernel(x, base_weight, lora_a_stacked, lora_b_stacked, base_indices,
           embeddings_indices):
    b, s = x.shape
    n_tok = b * s
    xf = x.reshape(n_tok).astype(jnp.int32)
    lidx = (embeddings_indices[1][:n_tok]
            // (_ORG_VOCAB + _EXTRA_VOCAB)).astype(jnp.int32)

    at3 = jnp.transpose(lora_a_stacked, (0, 2, 1))
    bt2 = lora_b_stacked[:, 0].transpose(0, 2, 1).reshape(
        _MAX_LORAS * _RANK, _EMBED_DIM).astype(jnp.bfloat16)

    rows, a_t = _sc_gather_build(n_tok)(base_weight, at3, xf, lidx)

    bt_tok = 512
    grid = (n_tok // bt_tok,)
    idx2 = base_indices[:n_tok].reshape(n_tok, 1).astype(jnp.int32)
    out = pl.pallas_call(
        _tc_body,
        grid=grid,
        in_specs=[
            pl.BlockSpec((bt_tok, _EMBED_DIM), lambda i: (i, 0)),
            pl.BlockSpec((_RANK, bt_tok), lambda i: (0, i)),
            pl.BlockSpec((bt_tok, 1), lambda i: (i, 0)),
            pl.BlockSpec((_MAX_LORAS * _RANK, _EMBED_DIM), lambda i: (0, 0)),
        ],
        out_specs=pl.BlockSpec((bt_tok, _EMBED_DIM), lambda i: (i, 0)),
        out_shape=jax.ShapeDtypeStruct((n_tok, _EMBED_DIM), jnp.float32),
    )(rows, a_t, idx2, bt2)

    return out.reshape(b, s, _EMBED_DIM)


# speedup vs baseline: 3.2671x; 1.0097x over previous
import functools

import jax
import jax.numpy as jnp
from jax import lax
from jax.experimental import pallas as pl
from jax.experimental.pallas import tpu as pltpu
from jax.experimental.pallas import tpu_sc as plsc

_ORG_VOCAB = 100000
_EXTRA_VOCAB = 256
_EMBED_DIM = 2048
_MAX_LORAS = 8
_RANK = 16

_NC, _NS = 2, 16
_NW = _NC * _NS
_CHUNK = 8
_AGRP = 8


def _sc_gather_build(n_tok: int):
    tpw = n_tok // _NW
    nch = tpw // _CHUNK
    ngr = tpw // _AGRP
    assert ngr == nch
    mesh = plsc.VectorSubcoreMesh(core_axis_name="c", subcore_axis_name="s")

    @functools.partial(
        pl.kernel,
        out_type=[
            jax.ShapeDtypeStruct((n_tok, _EMBED_DIM), jnp.float32),
            jax.ShapeDtypeStruct((_RANK, n_tok), jnp.float32),
        ],
        mesh=mesh,
        compiler_params=pltpu.CompilerParams(needs_layout_passes=False),
        scratch_types=[
            pltpu.VMEM((tpw,), jnp.int32),
            pltpu.VMEM((tpw,), jnp.int32),
            pltpu.VMEM((_RANK, tpw), jnp.float32),
            pltpu.VMEM((_CHUNK, _EMBED_DIM), jnp.float32),
            pltpu.VMEM((_CHUNK, _EMBED_DIM), jnp.float32),
            pltpu.VMEM((_AGRP, _RANK, 128), jnp.float32),
            pltpu.VMEM((_AGRP, _RANK, 128), jnp.float32),
            pltpu.SemaphoreType.DMA,
            pltpu.SemaphoreType.DMA,
            pltpu.SemaphoreType.DMA,
            pltpu.SemaphoreType.DMA,
        ],
    )
    def sc_gather(base_hbm, at_hbm, idx_hbm, lidx_hbm, rows_out, at_out,
                  idx_v, lidx_v, a_vt, buf0, buf1, slab0, slab1,
                  sem0, sem1, sema0, sema1):
        wid = lax.axis_index("s") * _NC + lax.axis_index("c")
        base = wid * tpw
        pltpu.sync_copy(idx_hbm.at[pl.ds(base, tpw)], idx_v)
        pltpu.sync_copy(lidx_hbm.at[pl.ds(base, tpw)], lidx_v)
        row = lax.iota(jnp.int32, 16)
        slabs = (slab0, slab1)
        semas = (sema0, sema1)

        bufs = (buf0, buf1)
        sems = (sem0, sem1)

        def fire_base(c, k):
            pltpu.async_copy(
                base_hbm.at[idx_v.at[pl.ds(c * _CHUNK, _CHUNK)]],
                bufs[k], sems[k])

        def wait_base(k):
            pltpu.make_async_copy(
                base_hbm.at[idx_v.at[pl.ds(0, _CHUNK)]],
                bufs[k], sems[k]).wait()

        def fire_a(g, k):
            off = (g // 2) * 16
            l_vec = lidx_v[pl.ds(off, 16)]
            v_vec = idx_v[pl.ds(off, 16)]
            for j in range(_AGRP):
                jj = j + _AGRP * k
                v0 = pl.multiple_of((v_vec[jj] >> 7) << 7, 128)
                pltpu.async_copy(
                    at_hbm.at[l_vec[jj], :, pl.ds(v0, 128)],
                    slabs[k].at[j], semas[k])

        def wait_a(k):
            for j in range(_AGRP):
                pltpu.make_async_copy(
                    at_hbm.at[0, :, pl.ds(0, 128)],
                    slabs[k].at[j], semas[k]).wait()

        def extract_a(g, k):
            v_vec = idx_v[pl.ds((g // 2) * 16, 16)]
            for j in range(_AGRP):
                jj = j + _AGRP * k
                t = g * _AGRP + j
                col = jnp.full((16,), v_vec[jj] & 127, jnp.int32)
                val = plsc.load_gather(
                    slabs[k], [jnp.full((16,), j, jnp.int32), row, col])
                plsc.store_scatter(
                    a_vt, [row, jnp.full((16,), t, jnp.int32)], val)

        half = nch // 2
        fire_base(0, 0)
        fire_a(0, 0)

        def body(i, carry):
            c0 = 2 * i
            fire_base(c0 + 1, 1)
            fire_a(c0 + 1, 1)
            wait_base(0)
            pltpu.sync_copy(bufs[0],
                            rows_out.at[pl.ds(base + c0 * _CHUNK, _CHUNK)])
            wait_a(0)
            extract_a(c0, 0)

            @pl.when(i + 1 < half)
            def _():
                fire_base(c0 + 2, 0)
                fire_a(c0 + 2, 0)

            wait_base(1)
            pltpu.sync_copy(
                bufs[1],
                rows_out.at[pl.ds(base + (c0 + 1) * _CHUNK, _CHUNK)])
            wait_a(1)
            extract_a(c0 + 1, 1)
            return carry

        lax.fori_loop(0, half, body, 0)
        pltpu.sync_copy(a_vt, at_out.at[:, pl.ds(base, tpw)])

    return sc_gather


def _tc_body(rows_ref, at_ref, idx_ref, bt_ref, out_ref):
    a = jnp.transpose(at_ref[...], (1, 0))
    idx = idx_ref[...]
    bt_blk = a.shape[0]
    cols = lax.broadcasted_iota(jnp.int32, (bt_blk, _MAX_LORAS * _RANK), 1)
    sel = (cols // _RANK) == idx
    a_exp = jnp.where(sel, jnp.concatenate([a] * _MAX_LORAS, axis=1), 0.0)
    delta = jnp.dot(a_exp.astype(jnp.bfloat16), bt_ref[...],
                    preferred_element_type=jnp.float32)
    out_ref[...] = rows_ref[...] + delta


def kernel(x, base_weight, lora_a_stacked, lora_b_stacked, base_indices,
           embeddings_indices):
    b, s = x.shape
    n_tok = b * s
    xf = x.reshape(n_tok).astype(jnp.int32)
    lidx = (embeddings_indices[1][:n_tok]
            // (_ORG_VOCAB + _EXTRA_VOCAB)).astype(jnp.int32)

    at3 = jnp.transpose(lora_a_stacked, (0, 2, 1))
    bt2 = lora_b_stacked[:, 0].transpose(0, 2, 1).reshape(
        _MAX_LORAS * _RANK, _EMBED_DIM).astype(jnp.bfloat16)

    rows, a_t = _sc_gather_build(n_tok)(base_weight, at3, xf, lidx)

    bt_tok = 1024
    grid = (n_tok // bt_tok,)
    idx2 = base_indices[:n_tok].reshape(n_tok, 1).astype(jnp.int32)
    out = pl.pallas_call(
        _tc_body,
        grid=grid,
        in_specs=[
            pl.BlockSpec((bt_tok, _EMBED_DIM), lambda i: (i, 0)),
            pl.BlockSpec((_RANK, bt_tok), lambda i: (0, i)),
            pl.BlockSpec((bt_tok, 1), lambda i: (i, 0)),
            pl.BlockSpec((_MAX_LORAS * _RANK, _EMBED_DIM), lambda i: (0, 0)),
        ],
        out_specs=pl.BlockSpec((bt_tok, _EMBED_DIM), lambda i: (i, 0)),
        out_shape=jax.ShapeDtypeStruct((n_tok, _EMBED_DIM), jnp.float32),
    )(rows, a_t, idx2, bt2)

    return out.reshape(b, s, _EMBED_DIM)
